# Initial kernel scaffold; baseline (speedup 1.0000x reference)
import jax
import jax.numpy as jnp
from jax.experimental import pallas as pl


def kernel(pred, target):
    raise NotImplementedError("write your pallas kernel here")


# speedup vs baseline: 2.0828x; 2.0828x over previous
import functools

import jax
import jax.numpy as jnp
from jax.experimental import pallas as pl

_SMOOTHING = 0.1
_IGNORE_INDEX = 0


def _loss_rows_kernel(pred_ref, tgt_ref, out_ref, *, vocab):
    x = pred_ref[...]
    t = tgt_ref[...]
    m = jnp.max(x, axis=-1, keepdims=True)
    s = jnp.sum(jnp.exp(x - m), axis=-1, keepdims=True)
    lse = m + jnp.log(s)
    sum_x = jnp.sum(x, axis=-1, keepdims=True)
    lane = jax.lax.broadcasted_iota(jnp.int32, x.shape, 1)
    pred_t = jnp.sum(jnp.where(lane == t, x, 0.0), axis=-1, keepdims=True)
    eps = _SMOOTHING / (vocab - 2)
    conf = 1.0 - _SMOOTHING
    sum_logp = sum_x - vocab * lse
    logp_t = pred_t - lse
    loss = -(eps * sum_logp + (conf - eps) * logp_t)
    out_ref[...] = jnp.where(t == _IGNORE_INDEX, 0.0, loss)


def kernel(pred, target):
    n, vocab = pred.shape
    rows_per_block = 32
    tgt = target.astype(jnp.int32).reshape(n, 1)
    row_losses = pl.pallas_call(
        functools.partial(_loss_rows_kernel, vocab=vocab),
        grid=(n // rows_per_block,),
        in_specs=[
            pl.BlockSpec((rows_per_block, vocab), lambda i: (i, 0)),
            pl.BlockSpec((rows_per_block, 1), lambda i: (i, 0)),
        ],
        out_specs=pl.BlockSpec((rows_per_block, 1), lambda i: (i, 0)),
        out_shape=jax.ShapeDtypeStruct((n, 1), jnp.float32),
    )(pred, tgt)
    return jnp.sum(row_losses) / n
